# baseline (speedup 1.0000x reference)
import functools

import jax
import jax.numpy as jnp
from jax import lax
from jax.experimental import pallas as pl
from jax.experimental.pallas import tpu as pltpu
from jax.experimental.pallas import tpu_sc as plsc

N = 10000
C = 128
E = 320000

NW = 32
NPAD = 10240
CHUNK = 128
KB = 1
NCHUNK = 80
NOP = NCHUNK // KB
OPW = 8
NWIN = NOP // OPW
E_PER_W = NCHUNK * CHUNK
EPAD = NW * E_PER_W
ROWS_PER_TILE = NPAD // 16

_MESH = plsc.VectorSubcoreMesh(core_axis_name="c", subcore_axis_name="s")


@functools.partial(
    pl.kernel,
    out_type=jax.ShapeDtypeStruct((NW, NPAD), jnp.float32),
    mesh=_MESH,
    scratch_types=[
        pltpu.VMEM((1024,), jnp.int32),
        pltpu.VMEM((NPAD,), jnp.float32),
    ],
    compiler_params=pltpu.CompilerParams(needs_layout_passes=False),
)
def _deg_kernel(dst_hbm, degp_hbm, dst_v, deg_v):
    c = lax.axis_index("c")
    s = lax.axis_index("s")
    wid = s * 2 + c

    zero16 = jnp.zeros((16,), jnp.float32)

    def zbody(i, carry):
        deg_v[pl.ds(i * 16, 16)] = zero16
        return carry

    lax.fori_loop(0, NPAD // 16, zbody, 0)

    ones16 = jnp.ones((16,), jnp.float32)

    def wbody(w, carry):
        pltpu.sync_copy(dst_hbm.at[pl.ds(wid * E_PER_W + w * 1024, 1024)],
                        dst_v)

        def body(i, carry2):
            idx = dst_v[pl.ds(i * 16, 16)]
            plsc.addupdate_scatter(deg_v, [idx], ones16)
            return carry2

        lax.fori_loop(0, 1024 // 16, body, 0)
        return carry

    lax.fori_loop(0, E_PER_W // 1024, wbody, 0)
    pltpu.sync_copy(deg_v, degp_hbm.at[wid])


def _mm_body(x_ref, w_ref, degp_ref, g_ref, dis_ref):
    h = jnp.dot(x_ref[...], w_ref[...], preferred_element_type=jnp.float32)
    deg = jnp.sum(degp_ref[...], axis=0) + 1.0
    dis = lax.rsqrt(deg)
    g_ref[...] = h * dis[:, None]
    dis_ref[...] = dis[:, None]


def _matmul_norm(x_pad, W, degp):
    BM = 256
    return pl.pallas_call(
        _mm_body,
        grid=(NPAD // BM,),
        in_specs=[
            pl.BlockSpec((BM, C), lambda i: (i, 0)),
            pl.BlockSpec((C, C), lambda i: (0, 0)),
            pl.BlockSpec((NW, BM), lambda i: (0, i)),
        ],
        out_specs=[
            pl.BlockSpec((BM, C), lambda i: (i, 0)),
            pl.BlockSpec((BM, 1), lambda i: (i, 0)),
        ],
        out_shape=[
            jax.ShapeDtypeStruct((NPAD, C), jnp.float32),
            jax.ShapeDtypeStruct((NPAD, 1), jnp.float32),
        ],
    )(x_pad, W, degp)


@functools.partial(
    pl.kernel,
    out_type=jax.ShapeDtypeStruct((2, NPAD, C), jnp.float32),
    mesh=_MESH,
    scratch_types=[
        pltpu.VMEM((NOP, CHUNK), jnp.int32),
        pltpu.VMEM((2, CHUNK), jnp.int32),
        pltpu.VMEM((2, CHUNK), jnp.int32),
        pltpu.VMEM((KB * CHUNK, C), jnp.float32),
        pltpu.VMEM((KB * CHUNK, C), jnp.float32),
        pltpu.VMEM_SHARED((NPAD, C), jnp.float32),
        pltpu.SemaphoreType.DMA,
        pltpu.SemaphoreType.DMA,
        pltpu.SemaphoreType.DMA,
        pltpu.SemaphoreType.DMA,
    ],
    compiler_params=pltpu.CompilerParams(needs_layout_passes=False),
)
def _edge_kernel(pk_hbm, g_hbm, outp_hbm, pk_w, src_c, dst_c, buf_a, buf_b,
                 S_sh, gsem_a, gsem_b, ssem_a, ssem_b):
    c = lax.axis_index("c")
    s = lax.axis_index("s")
    wid = s * 2 + c

    bufs = (buf_a, buf_b)
    gsems = (gsem_a, gsem_b)
    ssems = (ssem_a, ssem_b)

    pltpu.sync_copy(pk_hbm.at[wid], pk_w)

    zero16 = jnp.zeros((16,), jnp.float32)

    def zbody(i, carry):
        for j in range(C // 16):
            buf_a[i, pl.ds(j * 16, 16)] = zero16
            buf_b[i, pl.ds(j * 16, 16)] = zero16
        return carry

    lax.fori_loop(0, KB * CHUNK, zbody, 0)

    base0 = s * ROWS_PER_TILE
    for k in range(ROWS_PER_TILE // (KB * CHUNK)):
        pltpu.sync_copy(buf_a, S_sh.at[pl.ds(base0 + k * KB * CHUNK,
                                             KB * CHUNK)])

    mask14 = jnp.full((16,), (1 << 14) - 1, jnp.int32)

    def unpack(k, slot):
        for v in range(CHUNK // 16):
            pk = pk_w[k, pl.ds(v * 16, 16)]
            src_c[slot, pl.ds(v * 16, 16)] = lax.shift_right_logical(pk, 14)
            dst_c[slot, pl.ds(v * 16, 16)] = lax.bitwise_and(pk, mask14)

    def unpack_src(k, slot):
        for v in range(CHUNK // 16):
            pk = pk_w[k, pl.ds(v * 16, 16)]
            src_c[slot, pl.ds(v * 16, 16)] = lax.shift_right_logical(pk, 14)

    def unpack_dst(k, slot):
        for v in range(CHUNK // 16):
            pk = pk_w[k, pl.ds(v * 16, 16)]
            dst_c[slot, pl.ds(v * 16, 16)] = lax.bitwise_and(pk, mask14)

    def start_gather(slot, p):
        pltpu.async_copy(g_hbm.at[src_c.at[slot]], bufs[p], gsems[p])

    def wait_gather(p):
        pltpu.make_async_copy(g_hbm.at[src_c.at[0]], bufs[p],
                              gsems[p]).wait()

    def start_scatter(slot, p):
        pltpu.async_copy(bufs[p], S_sh.at[dst_c.at[slot]], ssems[p],
                         add=True)

    def wait_scatter(p):
        pltpu.make_async_copy(bufs[p], S_sh.at[dst_c.at[0]],
                              ssems[p]).wait()

    unpack(0, 0)
    start_gather(0, 0)
    plsc.subcore_barrier()
    start_scatter(0, 1)

    def step(j, p):
        unpack_src(j + 1, 1 - p)
        wait_gather(p)
        start_scatter(p, p)
        wait_scatter(1 - p)
        unpack_dst(j + 1, 1 - p)
        start_gather(1 - p, 1 - p)

    step(0, 0)

    def pbody(i, carry):
        j = 1 + i * 2
        step(j, 1)
        step(j + 1, 0)
        return carry

    lax.fori_loop(0, (NOP - 2) // 2, pbody, 0)

    wait_gather(1)
    start_scatter(1, 1)
    wait_scatter(0)
    wait_scatter(1)
    plsc.subcore_barrier()

    for k in range(ROWS_PER_TILE // (KB * CHUNK)):
        base = base0 + k * KB * CHUNK
        pltpu.sync_copy(S_sh.at[pl.ds(base, KB * CHUNK)], buf_a)
        pltpu.sync_copy(buf_a, outp_hbm.at[c, pl.ds(base, KB * CHUNK)])


def _fin_body(s0_ref, s1_ref, g_ref, dis_ref, b_ref, o_ref):
    t = (s0_ref[...] + s1_ref[...] + g_ref[...]) * dis_ref[...]
    o_ref[...] = jnp.maximum(t + b_ref[...], 0.0)


def _finish(S0, S1, g, dis, b2):
    BM = 256
    return pl.pallas_call(
        _fin_body,
        grid=(NPAD // BM,),
        in_specs=[
            pl.BlockSpec((BM, C), lambda i: (i, 0)),
            pl.BlockSpec((BM, C), lambda i: (i, 0)),
            pl.BlockSpec((BM, C), lambda i: (i, 0)),
            pl.BlockSpec((BM, 1), lambda i: (i, 0)),
            pl.BlockSpec((1, C), lambda i: (0, 0)),
        ],
        out_specs=pl.BlockSpec((BM, C), lambda i: (i, 0)),
        out_shape=jax.ShapeDtypeStruct((NPAD, C), jnp.float32),
    )(S0, S1, g, dis, b2)


def kernel(x, edge_index, W, b):
    ei = edge_index.astype(jnp.int32)
    pad = N + (jnp.arange(EPAD - E, dtype=jnp.int32) % (NPAD - N))
    src_p = jnp.concatenate([ei[0], pad])
    dst_p = jnp.concatenate([ei[1], pad])
    pk3 = ((src_p << 14) | dst_p).reshape(NW, NOP, CHUNK)

    x_pad = jnp.pad(x, ((0, NPAD - N), (0, 0)))

    degp = _deg_kernel(dst_p)
    g, dis = _matmul_norm(x_pad, W, degp)
    Sp = _edge_kernel(pk3, g)
    out = _finish(Sp[0], Sp[1], g, dis, b.reshape(1, C))
    return out[:N]


# speedup vs baseline: 1.0109x; 1.0109x over previous
import functools

import jax
import jax.numpy as jnp
from jax import lax
from jax.experimental import pallas as pl
from jax.experimental.pallas import tpu as pltpu
from jax.experimental.pallas import tpu_sc as plsc

N = 10000
C = 128
E = 320000

NW = 32
NPAD = 10240
CHUNK = 128
KB = 1
NCHUNK = 80
NOP = NCHUNK // KB
OPW = 8
NWIN = NOP // OPW
E_PER_W = NCHUNK * CHUNK
EPAD = NW * E_PER_W
ROWS_PER_TILE = NPAD // 16

_MESH = plsc.VectorSubcoreMesh(core_axis_name="c", subcore_axis_name="s")


@functools.partial(
    pl.kernel,
    out_type=jax.ShapeDtypeStruct((NW, NPAD), jnp.float32),
    mesh=_MESH,
    scratch_types=[
        pltpu.VMEM((1024,), jnp.int32),
        pltpu.VMEM((NPAD,), jnp.float32),
    ],
    compiler_params=pltpu.CompilerParams(needs_layout_passes=False),
)
def _deg_kernel(dst_hbm, degp_hbm, dst_v, deg_v):
    c = lax.axis_index("c")
    s = lax.axis_index("s")
    wid = s * 2 + c

    zero16 = jnp.zeros((16,), jnp.float32)

    def zbody(i, carry):
        deg_v[pl.ds(i * 16, 16)] = zero16
        return carry

    lax.fori_loop(0, NPAD // 16, zbody, 0)

    ones16 = jnp.ones((16,), jnp.float32)

    def wbody(w, carry):
        pltpu.sync_copy(dst_hbm.at[pl.ds(wid * E_PER_W + w * 1024, 1024)],
                        dst_v)

        def body(i, carry2):
            idx = dst_v[pl.ds(i * 16, 16)]
            plsc.addupdate_scatter(deg_v, [idx], ones16)
            return carry2

        lax.fori_loop(0, 1024 // 16, body, 0)
        return carry

    lax.fori_loop(0, E_PER_W // 1024, wbody, 0)
    pltpu.sync_copy(deg_v, degp_hbm.at[wid])


def _mm_body(x_ref, w_ref, degp_ref, g_ref, dis_ref):
    h = jnp.dot(x_ref[...], w_ref[...], preferred_element_type=jnp.float32)
    deg = jnp.sum(degp_ref[...], axis=0) + 1.0
    dis = lax.rsqrt(deg)
    g_ref[...] = h * dis[:, None]
    dis_ref[...] = dis[:, None]


def _matmul_norm(x_pad, W, degp):
    BM = 256
    return pl.pallas_call(
        _mm_body,
        grid=(NPAD // BM,),
        in_specs=[
            pl.BlockSpec((BM, C), lambda i: (i, 0)),
            pl.BlockSpec((C, C), lambda i: (0, 0)),
            pl.BlockSpec((NW, BM), lambda i: (0, i)),
        ],
        out_specs=[
            pl.BlockSpec((BM, C), lambda i: (i, 0)),
            pl.BlockSpec((BM, 1), lambda i: (i, 0)),
        ],
        out_shape=[
            jax.ShapeDtypeStruct((NPAD, C), jnp.float32),
            jax.ShapeDtypeStruct((NPAD, 1), jnp.float32),
        ],
    )(x_pad, W, degp)


@functools.partial(
    pl.kernel,
    out_type=jax.ShapeDtypeStruct((2, NPAD, C), jnp.float32),
    mesh=_MESH,
    scratch_types=[
        pltpu.VMEM((NOP, CHUNK), jnp.int32),
        pltpu.VMEM((2, CHUNK), jnp.int32),
        pltpu.VMEM((2, CHUNK), jnp.int32),
        pltpu.VMEM((KB * CHUNK, C), jnp.float32),
        pltpu.VMEM((KB * CHUNK, C), jnp.float32),
        pltpu.VMEM_SHARED((NPAD, C), jnp.float32),
        pltpu.SemaphoreType.DMA,
        pltpu.SemaphoreType.DMA,
    ],
    compiler_params=pltpu.CompilerParams(needs_layout_passes=False),
)
def _edge_kernel(pk_hbm, g_hbm, outp_hbm, pk_w, src_c, dst_c, buf_a, buf_b,
                 S_sh, gsem_a, gsem_b):
    c = lax.axis_index("c")
    s = lax.axis_index("s")
    wid = s * 2 + c

    bufs = (buf_a, buf_b)
    gsems = (gsem_a, gsem_b)

    pltpu.sync_copy(pk_hbm.at[wid], pk_w)

    zero16 = jnp.zeros((16,), jnp.float32)

    def zbody(i, carry):
        for j in range(C // 16):
            buf_a[i, pl.ds(j * 16, 16)] = zero16
            buf_b[i, pl.ds(j * 16, 16)] = zero16
        return carry

    lax.fori_loop(0, KB * CHUNK, zbody, 0)

    base0 = s * ROWS_PER_TILE
    for k in range(ROWS_PER_TILE // (KB * CHUNK)):
        pltpu.sync_copy(buf_a, S_sh.at[pl.ds(base0 + k * KB * CHUNK,
                                             KB * CHUNK)])

    mask14 = jnp.full((16,), (1 << 14) - 1, jnp.int32)

    def unpack(k, slot):
        for v in range(CHUNK // 16):
            pk = pk_w[k, pl.ds(v * 16, 16)]
            src_c[slot, pl.ds(v * 16, 16)] = lax.shift_right_logical(pk, 14)
            dst_c[slot, pl.ds(v * 16, 16)] = lax.bitwise_and(pk, mask14)

    def start_gather(slot, p):
        pltpu.async_copy(g_hbm.at[src_c.at[slot]], bufs[p], gsems[p])

    def wait_gather(p):
        pltpu.make_async_copy(g_hbm.at[src_c.at[0]], bufs[p],
                              gsems[p]).wait()

    def scatter(slot, p):
        pltpu.sync_copy(bufs[p], S_sh.at[dst_c.at[slot]], add=True)

    unpack(0, 0)
    start_gather(0, 0)
    plsc.subcore_barrier()

    def step(j, p):
        unpack(j + 1, 1 - p)
        wait_gather(p)
        start_gather(1 - p, 1 - p)
        scatter(p, p)

    step(0, 0)

    def pbody(i, carry):
        j = 1 + i * 2
        step(j, 1)
        step(j + 1, 0)
        return carry

    lax.fori_loop(0, (NOP - 2) // 2, pbody, 0)

    wait_gather(1)
    scatter(1, 1)
    plsc.subcore_barrier()

    for k in range(ROWS_PER_TILE // (KB * CHUNK)):
        base = base0 + k * KB * CHUNK
        pltpu.sync_copy(S_sh.at[pl.ds(base, KB * CHUNK)], buf_a)
        pltpu.sync_copy(buf_a, outp_hbm.at[c, pl.ds(base, KB * CHUNK)])


def _fin_body(s0_ref, s1_ref, g_ref, dis_ref, b_ref, o_ref):
    t = (s0_ref[...] + s1_ref[...] + g_ref[...]) * dis_ref[...]
    o_ref[...] = jnp.maximum(t + b_ref[...], 0.0)


def _finish(S0, S1, g, dis, b2):
    BM = 256
    return pl.pallas_call(
        _fin_body,
        grid=(NPAD // BM,),
        in_specs=[
            pl.BlockSpec((BM, C), lambda i: (i, 0)),
            pl.BlockSpec((BM, C), lambda i: (i, 0)),
            pl.BlockSpec((BM, C), lambda i: (i, 0)),
            pl.BlockSpec((BM, 1), lambda i: (i, 0)),
            pl.BlockSpec((1, C), lambda i: (0, 0)),
        ],
        out_specs=pl.BlockSpec((BM, C), lambda i: (i, 0)),
        out_shape=jax.ShapeDtypeStruct((NPAD, C), jnp.float32),
    )(S0, S1, g, dis, b2)


def kernel(x, edge_index, W, b):
    ei = edge_index.astype(jnp.int32)
    pad = N + (jnp.arange(EPAD - E, dtype=jnp.int32) % (NPAD - N))
    src_p = jnp.concatenate([ei[0], pad])
    dst_p = jnp.concatenate([ei[1], pad])
    pk3 = ((src_p << 14) | dst_p).reshape(NW, NOP, CHUNK)

    x_pad = jnp.pad(x, ((0, NPAD - N), (0, 0)))

    degp = _deg_kernel(dst_p)
    g, dis = _matmul_norm(x_pad, W, degp)
    Sp = _edge_kernel(pk3, g)
    out = _finish(Sp[0], Sp[1], g, dis, b.reshape(1, C))
    return out[:N]


# speedup vs baseline: 1.0204x; 1.0094x over previous
import functools

import jax
import jax.numpy as jnp
from jax import lax
from jax.experimental import pallas as pl
from jax.experimental.pallas import tpu as pltpu
from jax.experimental.pallas import tpu_sc as plsc

N = 10000
C = 128
E = 320000

NW = 32
NPAD = 10240
CHUNK = 128
KB = 1
NCHUNK = 80
NOP = NCHUNK // KB
OPW = 8
NWIN = NOP // OPW
E_PER_W = NCHUNK * CHUNK
EPAD = NW * E_PER_W
ROWS_PER_TILE = NPAD // 16

_MESH = plsc.VectorSubcoreMesh(core_axis_name="c", subcore_axis_name="s")


@functools.partial(
    pl.kernel,
    out_type=jax.ShapeDtypeStruct((NW, NPAD), jnp.float32),
    mesh=_MESH,
    scratch_types=[
        pltpu.VMEM((E_PER_W,), jnp.int32),
        pltpu.VMEM((NPAD,), jnp.float32),
    ],
    compiler_params=pltpu.CompilerParams(needs_layout_passes=False),
)
def _deg_kernel(dst_hbm, degp_hbm, dst_v, deg_v):
    c = lax.axis_index("c")
    s = lax.axis_index("s")
    wid = s * 2 + c

    zero16 = jnp.zeros((16,), jnp.float32)

    def zbody(i, carry):
        deg_v[pl.ds(i * 16, 16)] = zero16
        return carry

    lax.fori_loop(0, NPAD // 16, zbody, 0)

    pltpu.sync_copy(dst_hbm.at[pl.ds(wid * E_PER_W, E_PER_W)], dst_v)
    ones16 = jnp.ones((16,), jnp.float32)

    def body(i, carry):
        idx = dst_v[pl.ds(i * 16, 16)]
        plsc.addupdate_scatter(deg_v, [idx], ones16)
        return carry

    lax.fori_loop(0, E_PER_W // 16, body, 0)
    pltpu.sync_copy(deg_v, degp_hbm.at[wid])


def _mm_body(x_ref, w_ref, degp_ref, g_ref, dis_ref):
    h = jnp.dot(x_ref[...], w_ref[...], preferred_element_type=jnp.float32)
    deg = jnp.sum(degp_ref[...], axis=0) + 1.0
    dis = lax.rsqrt(deg)
    g_ref[...] = h * dis[:, None]
    dis_ref[...] = dis[:, None]


def _matmul_norm(x_pad, W, degp):
    BM = 256
    return pl.pallas_call(
        _mm_body,
        grid=(NPAD // BM,),
        in_specs=[
            pl.BlockSpec((BM, C), lambda i: (i, 0)),
            pl.BlockSpec((C, C), lambda i: (0, 0)),
            pl.BlockSpec((NW, BM), lambda i: (0, i)),
        ],
        out_specs=[
            pl.BlockSpec((BM, C), lambda i: (i, 0)),
            pl.BlockSpec((BM, 1), lambda i: (i, 0)),
        ],
        out_shape=[
            jax.ShapeDtypeStruct((NPAD, C), jnp.float32),
            jax.ShapeDtypeStruct((NPAD, 1), jnp.float32),
        ],
    )(x_pad, W, degp)


@functools.partial(
    pl.kernel,
    out_type=jax.ShapeDtypeStruct((2, NPAD, C), jnp.float32),
    mesh=_MESH,
    scratch_types=[
        pltpu.VMEM((NOP, CHUNK), jnp.int32),
        pltpu.VMEM((2, CHUNK), jnp.int32),
        pltpu.VMEM((2, CHUNK), jnp.int32),
        pltpu.VMEM((KB * CHUNK, C), jnp.float32),
        pltpu.VMEM((KB * CHUNK, C), jnp.float32),
        pltpu.VMEM_SHARED((NPAD, C), jnp.float32),
        pltpu.SemaphoreType.DMA,
        pltpu.SemaphoreType.DMA,
    ],
    compiler_params=pltpu.CompilerParams(needs_layout_passes=False),
)
def _edge_kernel(pk_hbm, g_hbm, outp_hbm, pk_w, src_c, dst_c, buf_a, buf_b,
                 S_sh, gsem_a, gsem_b):
    c = lax.axis_index("c")
    s = lax.axis_index("s")
    wid = s * 2 + c

    bufs = (buf_a, buf_b)
    gsems = (gsem_a, gsem_b)

    pltpu.sync_copy(pk_hbm.at[wid], pk_w)

    zero16 = jnp.zeros((16,), jnp.float32)

    def zbody(i, carry):
        for j in range(C // 16):
            buf_a[i, pl.ds(j * 16, 16)] = zero16
        return carry

    lax.fori_loop(0, KB * CHUNK, zbody, 0)

    base0 = s * ROWS_PER_TILE
    for k in range(ROWS_PER_TILE // (KB * CHUNK)):
        sl = pl.ds(base0 + k * KB * CHUNK, KB * CHUNK)

        @pl.when(c == 0)
        def _():
            pltpu.sync_copy(g_hbm.at[sl], S_sh.at[sl])

        @pl.when(c == 1)
        def _():
            pltpu.sync_copy(buf_a, S_sh.at[sl])

    mask14 = jnp.full((16,), (1 << 14) - 1, jnp.int32)

    def unpack(k, slot):
        for v in range(CHUNK // 16):
            pk = pk_w[k, pl.ds(v * 16, 16)]
            src_c[slot, pl.ds(v * 16, 16)] = lax.shift_right_logical(pk, 14)
            dst_c[slot, pl.ds(v * 16, 16)] = lax.bitwise_and(pk, mask14)

    def start_gather(slot, p):
        pltpu.async_copy(g_hbm.at[src_c.at[slot]], bufs[p], gsems[p])

    def wait_gather(p):
        pltpu.make_async_copy(g_hbm.at[src_c.at[0]], bufs[p],
                              gsems[p]).wait()

    def scatter(slot, p):
        pltpu.sync_copy(bufs[p], S_sh.at[dst_c.at[slot]], add=True)

    unpack(0, 0)
    start_gather(0, 0)
    plsc.subcore_barrier()

    def step(j, p):
        unpack(j + 1, 1 - p)
        wait_gather(p)
        start_gather(1 - p, 1 - p)
        scatter(p, p)

    step(0, 0)

    def pbody(i, carry):
        j = 1 + i * 2
        step(j, 1)
        step(j + 1, 0)
        return carry

    lax.fori_loop(0, (NOP - 2) // 2, pbody, 0)

    wait_gather(1)
    scatter(1, 1)
    plsc.subcore_barrier()

    for k in range(ROWS_PER_TILE // (KB * CHUNK)):
        base = base0 + k * KB * CHUNK
        pltpu.sync_copy(S_sh.at[pl.ds(base, KB * CHUNK)], buf_a)
        pltpu.sync_copy(buf_a, outp_hbm.at[c, pl.ds(base, KB * CHUNK)])


def _fin_body(s0_ref, s1_ref, dis_ref, b_ref, o_ref):
    t = (s0_ref[...] + s1_ref[...]) * dis_ref[...]
    o_ref[...] = jnp.maximum(t + b_ref[...], 0.0)


def _finish(S0, S1, dis, b2):
    BM = 256
    return pl.pallas_call(
        _fin_body,
        grid=(NPAD // BM,),
        in_specs=[
            pl.BlockSpec((BM, C), lambda i: (i, 0)),
            pl.BlockSpec((BM, C), lambda i: (i, 0)),
            pl.BlockSpec((BM, 1), lambda i: (i, 0)),
            pl.BlockSpec((1, C), lambda i: (0, 0)),
        ],
        out_specs=pl.BlockSpec((BM, C), lambda i: (i, 0)),
        out_shape=jax.ShapeDtypeStruct((NPAD, C), jnp.float32),
    )(S0, S1, dis, b2)


def kernel(x, edge_index, W, b):
    ei = edge_index.astype(jnp.int32)
    pad = N + (jnp.arange(EPAD - E, dtype=jnp.int32) % (NPAD - N))
    src_p = jnp.concatenate([ei[0], pad])
    dst_p = jnp.concatenate([ei[1], pad])
    pk3 = ((src_p << 14) | dst_p).reshape(NW, NOP, CHUNK)

    x_pad = jnp.pad(x, ((0, NPAD - N), (0, 0)))

    degp = _deg_kernel(dst_p)
    g, dis = _matmul_norm(x_pad, W, degp)
    Sp = _edge_kernel(pk3, g)
    out = _finish(Sp[0], Sp[1], dis, b.reshape(1, C))
    return out[:N]


# speedup vs baseline: 1.0226x; 1.0021x over previous
import functools

import jax
import jax.numpy as jnp
from jax import lax
from jax.experimental import pallas as pl
from jax.experimental.pallas import tpu as pltpu
from jax.experimental.pallas import tpu_sc as plsc

N = 10000
C = 128
E = 320000

NW = 32
NPAD = 10240
CHUNK = 128
KB = 1
NCHUNK = 80
NOP = NCHUNK // KB
E_PER_W = NCHUNK * CHUNK
EPAD = NW * E_PER_W
ROWS_PER_TILE = NPAD // 16

_MESH = plsc.VectorSubcoreMesh(core_axis_name="c", subcore_axis_name="s")


@functools.partial(
    pl.kernel,
    out_type=jax.ShapeDtypeStruct((NW, NPAD), jnp.float32),
    mesh=_MESH,
    scratch_types=[
        pltpu.VMEM((E_PER_W,), jnp.int32),
        pltpu.VMEM((NPAD,), jnp.float32),
    ],
    compiler_params=pltpu.CompilerParams(needs_layout_passes=False),
)
def _deg_kernel(dst_hbm, degp_hbm, dst_v, deg_v):
    c = lax.axis_index("c")
    s = lax.axis_index("s")
    wid = s * 2 + c

    zero16 = jnp.zeros((16,), jnp.float32)

    def zbody(i, carry):
        deg_v[pl.ds(i * 16, 16)] = zero16
        return carry

    lax.fori_loop(0, NPAD // 16, zbody, 0)

    pltpu.sync_copy(dst_hbm.at[pl.ds(wid * E_PER_W, E_PER_W)], dst_v)
    ones16 = jnp.ones((16,), jnp.float32)

    def body(i, carry):
        idx = dst_v[pl.ds(i * 16, 16)]
        plsc.addupdate_scatter(deg_v, [idx], ones16)
        return carry

    lax.fori_loop(0, E_PER_W // 16, body, 0)
    pltpu.sync_copy(deg_v, degp_hbm.at[wid])


def _mm_body(x_ref, w_ref, degp_ref, g_ref, dis_ref):
    h = jnp.dot(x_ref[...], w_ref[...], preferred_element_type=jnp.float32)
    deg = jnp.sum(degp_ref[...], axis=0) + 1.0
    dis = lax.rsqrt(deg)
    g_ref[...] = h * dis[:, None]
    dis_ref[...] = dis[:, None]


def _matmul_norm(x_pad, W, degp):
    BM = 256
    return pl.pallas_call(
        _mm_body,
        grid=(NPAD // BM,),
        in_specs=[
            pl.BlockSpec((BM, C), lambda i: (i, 0)),
            pl.BlockSpec((C, C), lambda i: (0, 0)),
            pl.BlockSpec((NW, BM), lambda i: (0, i)),
        ],
        out_specs=[
            pl.BlockSpec((BM, C), lambda i: (i, 0)),
            pl.BlockSpec((BM, 1), lambda i: (i, 0)),
        ],
        out_shape=[
            jax.ShapeDtypeStruct((NPAD, C), jnp.float32),
            jax.ShapeDtypeStruct((NPAD, 1), jnp.float32),
        ],
    )(x_pad, W, degp)


@functools.partial(
    pl.kernel,
    out_type=jax.ShapeDtypeStruct((2, NPAD, C), jnp.float32),
    mesh=_MESH,
    scratch_types=[
        pltpu.VMEM((NOP, CHUNK), jnp.int32),
        pltpu.VMEM((2, CHUNK), jnp.int32),
        pltpu.VMEM((2, CHUNK), jnp.int32),
        pltpu.VMEM((KB * CHUNK, C), jnp.float32),
        pltpu.VMEM((KB * CHUNK, C), jnp.float32),
        pltpu.VMEM_SHARED((NPAD, C), jnp.float32),
        pltpu.SemaphoreType.DMA,
        pltpu.SemaphoreType.DMA,
    ],
    compiler_params=pltpu.CompilerParams(needs_layout_passes=False),
)
def _edge_kernel(pk_hbm, g_hbm, outp_hbm, pk_w, src_c, dst_c, buf_a, buf_b,
                 S_sh, gsem_a, gsem_b):
    c = lax.axis_index("c")
    s = lax.axis_index("s")
    wid = s * 2 + c

    bufs = (buf_a, buf_b)
    gsems = (gsem_a, gsem_b)

    pltpu.sync_copy(pk_hbm.at[wid], pk_w)

    zero16 = jnp.zeros((16,), jnp.float32)

    def zbody(i, carry):
        for j in range(C // 16):
            buf_a[i, pl.ds(j * 16, 16)] = zero16
        return carry

    lax.fori_loop(0, KB * CHUNK, zbody, 0)

    base0 = s * ROWS_PER_TILE
    for k in range(ROWS_PER_TILE // (KB * CHUNK)):
        sl = pl.ds(base0 + k * KB * CHUNK, KB * CHUNK)

        @pl.when(c == 0)
        def _():
            pltpu.sync_copy(g_hbm.at[sl], S_sh.at[sl])

        @pl.when(c == 1)
        def _():
            pltpu.sync_copy(buf_a, S_sh.at[sl])

    mask14 = jnp.full((16,), (1 << 14) - 1, jnp.int32)

    def unpack(k, slot):
        for v in range(CHUNK // 16):
            pk = pk_w[k, pl.ds(v * 16, 16)]
            src_c[slot, pl.ds(v * 16, 16)] = lax.shift_right_logical(pk, 14)
            dst_c[slot, pl.ds(v * 16, 16)] = lax.bitwise_and(pk, mask14)

    def start_gather(slot, p):
        pltpu.async_copy(g_hbm.at[src_c.at[slot]], bufs[p], gsems[p])

    def wait_gather(p):
        pltpu.make_async_copy(g_hbm.at[src_c.at[0]], bufs[p],
                              gsems[p]).wait()

    def scatter(slot, p):
        pltpu.sync_copy(bufs[p], S_sh.at[dst_c.at[slot]], add=True)

    unpack(0, 0)
    start_gather(0, 0)
    plsc.subcore_barrier()

    def step(j, p):
        unpack(j + 1, 1 - p)
        wait_gather(p)
        start_gather(1 - p, 1 - p)
        scatter(p, p)

    step(0, 0)

    def pbody(i, carry):
        j = 1 + i * 2
        step(j, 1)
        step(j + 1, 0)
        return carry

    lax.fori_loop(0, (NOP - 2) // 2, pbody, 0)

    wait_gather(1)
    scatter(1, 1)
    plsc.subcore_barrier()

    for k in range(ROWS_PER_TILE // (KB * CHUNK)):
        base = base0 + k * KB * CHUNK
        pltpu.sync_copy(S_sh.at[pl.ds(base, KB * CHUNK)], buf_a)
        pltpu.sync_copy(buf_a, outp_hbm.at[c, pl.ds(base, KB * CHUNK)])


def _fin_body(s0_ref, s1_ref, dis_ref, b_ref, o_ref):
    t = (s0_ref[...] + s1_ref[...]) * dis_ref[...]
    o_ref[...] = jnp.maximum(t + b_ref[...], 0.0)


def _finish(S0, S1, dis, b2):
    BM = 256
    return pl.pallas_call(
        _fin_body,
        grid=(NPAD // BM,),
        in_specs=[
            pl.BlockSpec((BM, C), lambda i: (i, 0)),
            pl.BlockSpec((BM, C), lambda i: (i, 0)),
            pl.BlockSpec((BM, 1), lambda i: (i, 0)),
            pl.BlockSpec((1, C), lambda i: (0, 0)),
        ],
        out_specs=pl.BlockSpec((BM, C), lambda i: (i, 0)),
        out_shape=jax.ShapeDtypeStruct((NPAD, C), jnp.float32),
    )(S0, S1, dis, b2)


def kernel(x, edge_index, W, b):
    ei = edge_index.astype(jnp.int32)
    pad = N + (jnp.arange(EPAD - E, dtype=jnp.int32) % (NPAD - N))
    src_p = jnp.concatenate([ei[0], pad])
    dst_p = jnp.concatenate([ei[1], pad])
    pk3 = ((src_p << 14) | dst_p).reshape(NW, NOP, CHUNK)

    x_pad = jnp.pad(x, ((0, NPAD - N), (0, 0)))

    degp = _deg_kernel(dst_p)
    g, dis = _matmul_norm(x_pad, W, degp)
    Sp = _edge_kernel(pk3, g)
    out = _finish(Sp[0], Sp[1], dis, b.reshape(1, C))
    return out[:N]
